# Initial kernel scaffold; baseline (speedup 1.0000x reference)
import jax
import jax.numpy as jnp
from jax.experimental import pallas as pl


def kernel(x, pos, edge_index, edge_attr, params):
    raise NotImplementedError("write your pallas kernel here")


# speedup vs baseline: 3.1407x; 3.1407x over previous
import functools

import jax
import jax.numpy as jnp
from jax import lax
from jax.experimental import pallas as pl
from jax.experimental.pallas import tpu as pltpu
from jax.experimental.pallas import tpu_sc as plsc

F32 = jnp.float32
_NC, _NS = 2, 16
_NW = _NC * _NS


def _pick(total, cap):
    for c in range(cap - cap % 8, 7, -8):
        if total % c == 0:
            return c
    raise ValueError((total, cap))


def _leaky(x):
    return jnp.where(x > 0, x, 0.01 * x)


def _ln(h, g, b):
    m = jnp.mean(h, axis=-1, keepdims=True)
    v = jnp.mean((h - m) ** 2, axis=-1, keepdims=True)
    return (h - m) / jnp.sqrt(v + 1e-5) * g + b


def _dot(a, b):
    return jnp.dot(a, b, preferred_element_type=F32)


def _v(x):
    return x.reshape(1, -1)


def _tile(br, bc):
    return pl.BlockSpec((br, bc), lambda i: (i, 0))


def _const(br, bc):
    return pl.BlockSpec((br, bc), lambda i: (0, 0))


def _edge_stage(ga, gb, ef, w1c, p):
    E = ef.shape[0]
    BE = _pick(E, 2048)

    def body(ga_r, gb_r, ef_r, w1c_r, w2_r, w3_r, w4_r,
             b1_r, b2_r, b3_r, b4_r, g_r, bn_r, msg_o, efn_o):
        ef_t = ef_r[...]
        h = ga_r[...] + gb_r[...] + _dot(ef_t, w1c_r[...]) + b1_r[...]
        h = _leaky(h)
        h = _leaky(_dot(h, w2_r[...]) + b2_r[...])
        h = _leaky(_dot(h, w3_r[...]) + b3_r[...])
        h = _dot(h, w4_r[...]) + b4_r[...]
        msg = _ln(h, g_r[...], bn_r[...])
        msg_o[...] = msg
        efn_o[...] = ef_t + msg

    ws = [w1c, p["lin"][1]["w"], p["lin"][2]["w"], p["lin"][3]["w"]]
    bs = [_v(p["lin"][i]["b"]) for i in range(4)]
    return pl.pallas_call(
        body,
        grid=(E // BE,),
        in_specs=[_tile(BE, 128)] * 3 + [_const(128, 128)] * 4
                 + [_const(1, 128)] * 6,
        out_specs=[_tile(BE, 128), _tile(BE, 128)],
        out_shape=[jax.ShapeDtypeStruct((E, 128), F32)] * 2,
    )(ga, gb, ef, *ws, *bs, _v(p["ln"]["g"]), _v(p["ln"]["b"]))


def _node_stage(nf, agg2, p):
    N = nf.shape[0]
    BN = _pick(N, 2048)
    v1 = p["lin"][0]["w"]

    def body(nf_r, a_r, v1a_r, v1b_r, w2_r, w3_r, w4_r,
             b1_r, b2_r, b3_r, b4_r, g_r, bn_r, out_o):
        nf_t = nf_r[...]
        a = a_r[0] + a_r[1]
        h = _dot(nf_t, v1a_r[...]) + _dot(a, v1b_r[...]) + b1_r[...]
        h = _leaky(h)
        h = _leaky(_dot(h, w2_r[...]) + b2_r[...])
        h = _leaky(_dot(h, w3_r[...]) + b3_r[...])
        h = _dot(h, w4_r[...]) + b4_r[...]
        out_o[...] = nf_t + _ln(h, g_r[...], bn_r[...])

    ws = [v1[:128], v1[128:], p["lin"][1]["w"], p["lin"][2]["w"],
          p["lin"][3]["w"]]
    bs = [_v(p["lin"][i]["b"]) for i in range(4)]
    return pl.pallas_call(
        body,
        grid=(N // BN,),
        in_specs=[_tile(BN, 128),
                  pl.BlockSpec((2, BN, 128), lambda i: (0, i, 0))]
                 + [_const(128, 128)] * 5 + [_const(1, 128)] * 6,
        out_specs=_tile(BN, 128),
        out_shape=jax.ShapeDtypeStruct((N, 128), F32),
    )(nf, agg2, *ws, *bs, _v(p["ln"]["g"]), _v(p["ln"]["b"]))


def _prep_stage(nf, w1a, w1b):
    N = nf.shape[0]
    BN = _pick(N, 2048)

    def body(nf_r, wa_r, wb_r, pa_o, pb_o):
        nf_t = nf_r[...]
        pa_o[...] = _dot(nf_t, wa_r[...])
        pb_o[...] = _dot(nf_t, wb_r[...])

    return pl.pallas_call(
        body,
        grid=(N // BN,),
        in_specs=[_tile(BN, 128)] + [_const(128, 128)] * 2,
        out_specs=[_tile(BN, 128), _tile(BN, 128)],
        out_shape=[jax.ShapeDtypeStruct((N, 128), F32)] * 2,
    )(nf, w1a, w1b)


def _node_encoder(x2, posp, emb, p):
    N = x2.shape[0]
    BN = _pick(N, 2048)
    w1 = p["lin"][0]["w"]
    wemb = jnp.zeros((16, 128), F32).at[:9].set(_dot(emb, w1[:16]))
    wpos = jnp.zeros((16, 128), F32).at[:14].set(w1[16:30])

    def body(x_r, pos_r, wemb_r, wpos_r, w2_r, w3_r, w4_r,
             b1_r, b2_r, b3_r, b4_r, g_r, bn_r, out_o):
        oh = (x_r[...] == lax.broadcasted_iota(jnp.int32, (BN, 16), 1))
        h = (_dot(oh.astype(F32), wemb_r[...]) + _dot(pos_r[...], wpos_r[...])
             + b1_r[...])
        h = _leaky(h)
        h = _leaky(_dot(h, w2_r[...]) + b2_r[...])
        h = _leaky(_dot(h, w3_r[...]) + b3_r[...])
        h = _dot(h, w4_r[...]) + b4_r[...]
        out_o[...] = _ln(h, g_r[...], bn_r[...])

    ws = [p["lin"][1]["w"], p["lin"][2]["w"], p["lin"][3]["w"]]
    bs = [_v(p["lin"][i]["b"]) for i in range(4)]
    return pl.pallas_call(
        body,
        grid=(N // BN,),
        in_specs=[pl.BlockSpec((BN, 1), lambda i: (i, 0)), _tile(BN, 16),
                  _const(16, 128), _const(16, 128)]
                 + [_const(128, 128)] * 3 + [_const(1, 128)] * 6,
        out_specs=_tile(BN, 128),
        out_shape=jax.ShapeDtypeStruct((N, 128), F32),
    )(x2, posp, wemb, wpos, *ws, *bs, _v(p["ln"]["g"]), _v(p["ln"]["b"]))


def _edge_encoder(eap, p):
    E = eap.shape[0]
    BE = _pick(E, 2048)
    w1p = jnp.zeros((8, 128), F32).at[:3].set(p["lin"][0]["w"])

    def body(ea_r, w1_r, w2_r, w3_r, w4_r,
             b1_r, b2_r, b3_r, b4_r, g_r, bn_r, out_o):
        h = _dot(ea_r[...], w1_r[...]) + b1_r[...]
        h = _leaky(h)
        h = _leaky(_dot(h, w2_r[...]) + b2_r[...])
        h = _leaky(_dot(h, w3_r[...]) + b3_r[...])
        h = _dot(h, w4_r[...]) + b4_r[...]
        out_o[...] = _ln(h, g_r[...], bn_r[...])

    ws = [p["lin"][1]["w"], p["lin"][2]["w"], p["lin"][3]["w"]]
    bs = [_v(p["lin"][i]["b"]) for i in range(4)]
    return pl.pallas_call(
        body,
        grid=(E // BE,),
        in_specs=[_tile(BE, 8), _const(8, 128)] + [_const(128, 128)] * 3
                 + [_const(1, 128)] * 6,
        out_specs=_tile(BE, 128),
        out_shape=jax.ShapeDtypeStruct((E, 128), F32),
    )(eap, w1p, *ws, *bs, _v(p["ln"]["g"]), _v(p["ln"]["b"]))


def _decoder_stage(nf, p):
    N = nf.shape[0]
    BN = _pick(N, 2048)
    w4p = jnp.zeros((128, 128), F32).at[:, :2].set(p["lin"][3]["w"])
    b4p = jnp.zeros((128,), F32).at[:2].set(p["lin"][3]["b"])

    def body(nf_r, w1_r, w2_r, w3_r, w4_r, b1_r, b2_r, b3_r, b4_r, out_o):
        h = _leaky(_dot(nf_r[...], w1_r[...]) + b1_r[...])
        h = _leaky(_dot(h, w2_r[...]) + b2_r[...])
        h = _leaky(_dot(h, w3_r[...]) + b3_r[...])
        out_o[...] = _dot(h, w4_r[...]) + b4_r[...]

    return pl.pallas_call(
        body,
        grid=(N // BN,),
        in_specs=[_tile(BN, 128)] + [_const(128, 128)] * 4
                 + [_const(1, 128)] * 4,
        out_specs=_tile(BN, 128),
        out_shape=jax.ShapeDtypeStruct((N, 128), F32),
    )(nf, p["lin"][0]["w"], p["lin"][1]["w"], p["lin"][2]["w"], w4p,
      _v(p["lin"][0]["b"]), _v(p["lin"][1]["b"]), _v(p["lin"][2]["b"]),
      _v(b4p))


def _sc_mesh():
    return plsc.VectorSubcoreMesh(core_axis_name="c", subcore_axis_name="s",
                                  num_cores=_NC, num_subcores=_NS)


def _sc_gather(pa, pb, dst3, src3):
    _, CH, K = dst3.shape
    EW = CH * K
    E = _NW * EW

    @functools.partial(
        pl.kernel,
        out_type=(jax.ShapeDtypeStruct((E, 128), F32),
                  jax.ShapeDtypeStruct((E, 128), F32)),
        mesh=_sc_mesh(),
        scratch_types=[
            pltpu.VMEM((CH, K), jnp.int32),
            pltpu.VMEM((CH, K), jnp.int32),
            pltpu.VMEM((K, 128), F32),
            pltpu.VMEM((K, 128), F32),
            pltpu.SemaphoreType.DMA,
            pltpu.SemaphoreType.DMA,
        ],
    )
    def k(pa_h, pb_h, d_h, s_h, ga_h, gb_h, di, si, ra, rb, sa, sb):
        wid = lax.axis_index("s") * _NC + lax.axis_index("c")
        base = wid * EW
        pltpu.sync_copy(d_h.at[wid], di)
        pltpu.sync_copy(s_h.at[wid], si)

        def body(j, carry):
            off = base + j * K
            ca = pltpu.async_copy(pa_h.at[di.at[j]], ra, sa)
            cb = pltpu.async_copy(pb_h.at[si.at[j]], rb, sb)
            ca.wait()
            cb.wait()
            pltpu.sync_copy(ra, ga_h.at[pl.ds(off, K)])
            pltpu.sync_copy(rb, gb_h.at[pl.ds(off, K)])
            return carry

        lax.fori_loop(0, CH, body, 0)

    return k(pa, pb, dst3, src3)


def _sc_scatter(msg, dst3, zeros_n):
    _, CH, K = dst3.shape
    EW = CH * K
    N = zeros_n.shape[0]

    @functools.partial(
        pl.kernel,
        out_type=jax.ShapeDtypeStruct((_NC, N, 128), F32),
        mesh=_sc_mesh(),
        scratch_types=[
            pltpu.VMEM((CH, K), jnp.int32),
            pltpu.VMEM((K, 128), F32),
            pltpu.VMEM_SHARED((N, 128), F32),
        ],
    )
    def k(m_h, d_h, z_h, out_h, di, mb, acc):
        cid = lax.axis_index("c")
        sid = lax.axis_index("s")
        wid = sid * _NC + cid

        @pl.when(sid == 0)
        def _zero():
            pltpu.sync_copy(z_h, acc)

        plsc.subcore_barrier()
        pltpu.sync_copy(d_h.at[wid], di)

        def body(j, carry):
            off = wid * EW + j * K
            pltpu.sync_copy(m_h.at[pl.ds(off, K)], mb)
            pltpu.sync_copy(mb, acc.at[di.at[j]], add=True)
            return carry

        lax.fori_loop(0, CH, body, 0)
        plsc.subcore_barrier()

        @pl.when(sid == 0)
        def _dump():
            pltpu.sync_copy(acc, out_h.at[cid])

    return k(msg, dst3, zeros_n)


def kernel(x, pos, edge_index, edge_attr, params):
    N = pos.shape[0]
    E = edge_attr.shape[0]
    EW = E // _NW
    K = _pick(EW, 128)
    CH = EW // K

    src = edge_index[0].astype(jnp.int32)
    dst = edge_index[1].astype(jnp.int32)
    dst3 = dst.reshape(_NW, CH, K)
    src3 = src.reshape(_NW, CH, K)
    zeros_n = jnp.zeros((N, 128), F32)

    x2 = x.astype(jnp.int32).reshape(N, 1)
    posp = jnp.concatenate(
        [pos.astype(F32), jnp.zeros((N, 2), F32)], axis=1)
    eap = jnp.concatenate(
        [edge_attr.astype(F32), jnp.zeros((E, 5), F32)], axis=1)

    nf = _node_encoder(x2, posp, params["embedding"], params["node_in"])
    ef = _edge_encoder(eap, params["edge_in"])

    for lp in params["layers"]:
        w1 = lp["edge"]["lin"][0]["w"]
        pa, pb = _prep_stage(nf, w1[:128], w1[128:256])
        ga, gb = _sc_gather(pa, pb, dst3, src3)
        msg, ef = _edge_stage(ga, gb, ef, w1[256:], lp["edge"])
        agg2 = _sc_scatter(msg, dst3, zeros_n)
        nf = _node_stage(nf, agg2, lp["node"])

    out = _decoder_stage(nf, params["decoder"])
    return out[:, :2]


# speedup vs baseline: 3.5652x; 1.1352x over previous
import functools

import jax
import jax.numpy as jnp
from jax import lax
from jax.experimental import pallas as pl
from jax.experimental.pallas import tpu as pltpu
from jax.experimental.pallas import tpu_sc as plsc

F32 = jnp.float32
_NC, _NS = 2, 16
_NW = _NC * _NS


def _pick(total, cap):
    for c in range(cap - cap % 8, 7, -8):
        if total % c == 0:
            return c
    raise ValueError((total, cap))


def _leaky(x):
    return jnp.where(x > 0, x, 0.01 * x)


def _ln(h, g, b):
    m = jnp.mean(h, axis=-1, keepdims=True)
    v = jnp.mean((h - m) ** 2, axis=-1, keepdims=True)
    return (h - m) / jnp.sqrt(v + 1e-5) * g + b


def _dot(a, b):
    return jnp.dot(a, b, preferred_element_type=F32)


def _v(x):
    return x.reshape(1, -1)


def _tile(br, bc):
    return pl.BlockSpec((br, bc), lambda i: (i, 0))


def _const(br, bc):
    return pl.BlockSpec((br, bc), lambda i: (0, 0))


def _edge_stage(ga, gb, ef, w1c, p):
    E = ef.shape[0]
    BE = _pick(E, 2048)

    def body(ga_r, gb_r, ef_r, w1c_r, w2_r, w3_r, w4_r,
             b1_r, b2_r, b3_r, b4_r, g_r, bn_r, msg_o, efn_o):
        ef_t = ef_r[...]
        h = ga_r[...] + gb_r[...] + _dot(ef_t, w1c_r[...]) + b1_r[...]
        h = _leaky(h)
        h = _leaky(_dot(h, w2_r[...]) + b2_r[...])
        h = _leaky(_dot(h, w3_r[...]) + b3_r[...])
        h = _dot(h, w4_r[...]) + b4_r[...]
        msg = _ln(h, g_r[...], bn_r[...])
        msg_o[...] = msg
        efn_o[...] = ef_t + msg

    ws = [w1c, p["lin"][1]["w"], p["lin"][2]["w"], p["lin"][3]["w"]]
    bs = [_v(p["lin"][i]["b"]) for i in range(4)]
    return pl.pallas_call(
        body,
        grid=(E // BE,),
        in_specs=[_tile(BE, 128)] * 3 + [_const(128, 128)] * 4
                 + [_const(1, 128)] * 6,
        out_specs=[_tile(BE, 128), _tile(BE, 128)],
        out_shape=[jax.ShapeDtypeStruct((E, 128), F32)] * 2,
    )(ga, gb, ef, *ws, *bs, _v(p["ln"]["g"]), _v(p["ln"]["b"]))


def _node_stage(nf, agg2, p):
    N = nf.shape[0]
    BN = _pick(N, 2048)
    v1 = p["lin"][0]["w"]

    def body(nf_r, a_r, v1a_r, v1b_r, w2_r, w3_r, w4_r,
             b1_r, b2_r, b3_r, b4_r, g_r, bn_r, out_o):
        nf_t = nf_r[...]
        a = a_r[0] + a_r[1]
        h = _dot(nf_t, v1a_r[...]) + _dot(a, v1b_r[...]) + b1_r[...]
        h = _leaky(h)
        h = _leaky(_dot(h, w2_r[...]) + b2_r[...])
        h = _leaky(_dot(h, w3_r[...]) + b3_r[...])
        h = _dot(h, w4_r[...]) + b4_r[...]
        out_o[...] = nf_t + _ln(h, g_r[...], bn_r[...])

    ws = [v1[:128], v1[128:], p["lin"][1]["w"], p["lin"][2]["w"],
          p["lin"][3]["w"]]
    bs = [_v(p["lin"][i]["b"]) for i in range(4)]
    return pl.pallas_call(
        body,
        grid=(N // BN,),
        in_specs=[_tile(BN, 128),
                  pl.BlockSpec((2, BN, 128), lambda i: (0, i, 0))]
                 + [_const(128, 128)] * 5 + [_const(1, 128)] * 6,
        out_specs=_tile(BN, 128),
        out_shape=jax.ShapeDtypeStruct((N, 128), F32),
    )(nf, agg2, *ws, *bs, _v(p["ln"]["g"]), _v(p["ln"]["b"]))


def _prep_stage(nf, w1a, w1b):
    N = nf.shape[0]
    BN = _pick(N, 2048)

    def body(nf_r, wa_r, wb_r, pa_o, pb_o):
        nf_t = nf_r[...]
        pa_o[...] = _dot(nf_t, wa_r[...])
        pb_o[...] = _dot(nf_t, wb_r[...])

    return pl.pallas_call(
        body,
        grid=(N // BN,),
        in_specs=[_tile(BN, 128)] + [_const(128, 128)] * 2,
        out_specs=[_tile(BN, 128), _tile(BN, 128)],
        out_shape=[jax.ShapeDtypeStruct((N, 128), F32)] * 2,
    )(nf, w1a, w1b)


def _node_encoder(x2, posp, emb, p):
    N = x2.shape[0]
    BN = _pick(N, 2048)
    w1 = p["lin"][0]["w"]
    wemb = jnp.zeros((16, 128), F32).at[:9].set(_dot(emb, w1[:16]))
    wpos = jnp.zeros((16, 128), F32).at[:14].set(w1[16:30])

    def body(x_r, pos_r, wemb_r, wpos_r, w2_r, w3_r, w4_r,
             b1_r, b2_r, b3_r, b4_r, g_r, bn_r, out_o):
        oh = (x_r[...] == lax.broadcasted_iota(jnp.int32, (BN, 16), 1))
        h = (_dot(oh.astype(F32), wemb_r[...]) + _dot(pos_r[...], wpos_r[...])
             + b1_r[...])
        h = _leaky(h)
        h = _leaky(_dot(h, w2_r[...]) + b2_r[...])
        h = _leaky(_dot(h, w3_r[...]) + b3_r[...])
        h = _dot(h, w4_r[...]) + b4_r[...]
        out_o[...] = _ln(h, g_r[...], bn_r[...])

    ws = [p["lin"][1]["w"], p["lin"][2]["w"], p["lin"][3]["w"]]
    bs = [_v(p["lin"][i]["b"]) for i in range(4)]
    return pl.pallas_call(
        body,
        grid=(N // BN,),
        in_specs=[pl.BlockSpec((BN, 1), lambda i: (i, 0)), _tile(BN, 16),
                  _const(16, 128), _const(16, 128)]
                 + [_const(128, 128)] * 3 + [_const(1, 128)] * 6,
        out_specs=_tile(BN, 128),
        out_shape=jax.ShapeDtypeStruct((N, 128), F32),
    )(x2, posp, wemb, wpos, *ws, *bs, _v(p["ln"]["g"]), _v(p["ln"]["b"]))


def _edge_encoder(eap, p):
    E = eap.shape[0]
    BE = _pick(E, 2048)
    w1p = jnp.zeros((8, 128), F32).at[:3].set(p["lin"][0]["w"])

    def body(ea_r, w1_r, w2_r, w3_r, w4_r,
             b1_r, b2_r, b3_r, b4_r, g_r, bn_r, out_o):
        h = _dot(ea_r[...], w1_r[...]) + b1_r[...]
        h = _leaky(h)
        h = _leaky(_dot(h, w2_r[...]) + b2_r[...])
        h = _leaky(_dot(h, w3_r[...]) + b3_r[...])
        h = _dot(h, w4_r[...]) + b4_r[...]
        out_o[...] = _ln(h, g_r[...], bn_r[...])

    ws = [p["lin"][1]["w"], p["lin"][2]["w"], p["lin"][3]["w"]]
    bs = [_v(p["lin"][i]["b"]) for i in range(4)]
    return pl.pallas_call(
        body,
        grid=(E // BE,),
        in_specs=[_tile(BE, 8), _const(8, 128)] + [_const(128, 128)] * 3
                 + [_const(1, 128)] * 6,
        out_specs=_tile(BE, 128),
        out_shape=jax.ShapeDtypeStruct((E, 128), F32),
    )(eap, w1p, *ws, *bs, _v(p["ln"]["g"]), _v(p["ln"]["b"]))


def _decoder_stage(nf, p):
    N = nf.shape[0]
    BN = _pick(N, 2048)
    w4p = jnp.zeros((128, 128), F32).at[:, :2].set(p["lin"][3]["w"])
    b4p = jnp.zeros((128,), F32).at[:2].set(p["lin"][3]["b"])

    def body(nf_r, w1_r, w2_r, w3_r, w4_r, b1_r, b2_r, b3_r, b4_r, out_o):
        h = _leaky(_dot(nf_r[...], w1_r[...]) + b1_r[...])
        h = _leaky(_dot(h, w2_r[...]) + b2_r[...])
        h = _leaky(_dot(h, w3_r[...]) + b3_r[...])
        out_o[...] = _dot(h, w4_r[...]) + b4_r[...]

    return pl.pallas_call(
        body,
        grid=(N // BN,),
        in_specs=[_tile(BN, 128)] + [_const(128, 128)] * 4
                 + [_const(1, 128)] * 4,
        out_specs=_tile(BN, 128),
        out_shape=jax.ShapeDtypeStruct((N, 128), F32),
    )(nf, p["lin"][0]["w"], p["lin"][1]["w"], p["lin"][2]["w"], w4p,
      _v(p["lin"][0]["b"]), _v(p["lin"][1]["b"]), _v(p["lin"][2]["b"]),
      _v(b4p))


def _sc_mesh():
    return plsc.VectorSubcoreMesh(core_axis_name="c", subcore_axis_name="s",
                                  num_cores=_NC, num_subcores=_NS)


def _sc_gather(pa, pb, dst3, src3):
    _, CH, K = dst3.shape
    EW = CH * K
    E = _NW * EW

    @functools.partial(
        pl.kernel,
        out_type=(jax.ShapeDtypeStruct((E, 128), F32),
                  jax.ShapeDtypeStruct((E, 128), F32)),
        mesh=_sc_mesh(),
        scratch_types=[
            pltpu.VMEM((CH, K), jnp.int32),
            pltpu.VMEM((CH, K), jnp.int32),
            pltpu.VMEM((2, K, 128), F32),
            pltpu.VMEM((2, K, 128), F32),
            pltpu.SemaphoreType.DMA((2,)),
            pltpu.SemaphoreType.DMA((2,)),
            pltpu.SemaphoreType.DMA((2,)),
            pltpu.SemaphoreType.DMA((2,)),
        ],
    )
    def k(pa_h, pb_h, d_h, s_h, ga_h, gb_h, di, si, ra, rb,
          sga, sgb, swa, swb):
        wid = lax.axis_index("s") * _NC + lax.axis_index("c")
        base = wid * EW
        pltpu.sync_copy(d_h.at[wid], di)
        pltpu.sync_copy(s_h.at[wid], si)
        pltpu.async_copy(pa_h.at[di.at[0]], ra.at[0], sga.at[0])
        pltpu.async_copy(pb_h.at[si.at[0]], rb.at[0], sgb.at[0])

        def body(j, carry):
            cur = lax.rem(j, 2)
            nxt = 1 - cur
            off = base + j * K
            pltpu.make_async_copy(pa_h.at[di.at[j]], ra.at[cur],
                                  sga.at[cur]).wait()
            pltpu.make_async_copy(pb_h.at[si.at[j]], rb.at[cur],
                                  sgb.at[cur]).wait()
            pltpu.async_copy(ra.at[cur], ga_h.at[pl.ds(off, K)], swa.at[cur])
            pltpu.async_copy(rb.at[cur], gb_h.at[pl.ds(off, K)], swb.at[cur])

            @pl.when(j >= 1)
            def _wait_prev_writes():
                pltpu.make_async_copy(ra.at[nxt], ga_h.at[pl.ds(base, K)],
                                      swa.at[nxt]).wait()
                pltpu.make_async_copy(rb.at[nxt], gb_h.at[pl.ds(base, K)],
                                      swb.at[nxt]).wait()

            @pl.when(j + 1 < CH)
            def _start_next():
                pltpu.async_copy(pa_h.at[di.at[j + 1]], ra.at[nxt],
                                 sga.at[nxt])
                pltpu.async_copy(pb_h.at[si.at[j + 1]], rb.at[nxt],
                                 sgb.at[nxt])

            return carry

        lax.fori_loop(0, CH, body, 0)
        last = (CH - 1) % 2
        pltpu.make_async_copy(ra.at[last], ga_h.at[pl.ds(base, K)],
                              swa.at[last]).wait()
        pltpu.make_async_copy(rb.at[last], gb_h.at[pl.ds(base, K)],
                              swb.at[last]).wait()

    return k(pa, pb, dst3, src3)


def _sc_scatter(msg, dst3, zeros_n):
    _, CH, K = dst3.shape
    EW = CH * K
    N = zeros_n.shape[0]

    @functools.partial(
        pl.kernel,
        out_type=jax.ShapeDtypeStruct((_NC, N, 128), F32),
        mesh=_sc_mesh(),
        scratch_types=[
            pltpu.VMEM((CH, K), jnp.int32),
            pltpu.VMEM((2, K, 128), F32),
            pltpu.VMEM_SHARED((N, 128), F32),
            pltpu.SemaphoreType.DMA((2,)),
            pltpu.SemaphoreType.DMA((2,)),
        ],
    )
    def k(m_h, d_h, z_h, out_h, di, mb, acc, sml, ssc):
        cid = lax.axis_index("c")
        sid = lax.axis_index("s")
        wid = sid * _NC + cid
        base = wid * EW

        @pl.when(sid == 0)
        def _zero():
            pltpu.sync_copy(z_h, acc)

        plsc.subcore_barrier()
        pltpu.sync_copy(d_h.at[wid], di)
        pltpu.async_copy(m_h.at[pl.ds(base, K)], mb.at[0], sml.at[0])

        def body(j, carry):
            cur = lax.rem(j, 2)
            nxt = 1 - cur
            pltpu.make_async_copy(m_h.at[pl.ds(base, K)], mb.at[cur],
                                  sml.at[cur]).wait()
            pltpu.async_copy(mb.at[cur], acc.at[di.at[j]], ssc.at[cur],
                             add=True)

            @pl.when(j >= 1)
            def _wait_prev_scatter():
                pltpu.make_async_copy(mb.at[nxt], acc.at[di.at[0]],
                                      ssc.at[nxt]).wait()

            @pl.when(j + 1 < CH)
            def _start_next():
                pltpu.async_copy(m_h.at[pl.ds(base + (j + 1) * K, K)],
                                 mb.at[nxt], sml.at[nxt])

            return carry

        lax.fori_loop(0, CH, body, 0)
        last = (CH - 1) % 2
        pltpu.make_async_copy(mb.at[last], acc.at[di.at[0]],
                              ssc.at[last]).wait()
        plsc.subcore_barrier()

        @pl.when(sid == 0)
        def _dump():
            pltpu.sync_copy(acc, out_h.at[cid])

    return k(msg, dst3, zeros_n)


def kernel(x, pos, edge_index, edge_attr, params):
    N = pos.shape[0]
    E = edge_attr.shape[0]
    EW = E // _NW
    K = _pick(EW, 128)
    CH = EW // K

    src = edge_index[0].astype(jnp.int32)
    dst = edge_index[1].astype(jnp.int32)
    dst3 = dst.reshape(_NW, CH, K)
    src3 = src.reshape(_NW, CH, K)
    zeros_n = jnp.zeros((N, 128), F32)

    x2 = x.astype(jnp.int32).reshape(N, 1)
    posp = jnp.concatenate(
        [pos.astype(F32), jnp.zeros((N, 2), F32)], axis=1)
    eap = jnp.concatenate(
        [edge_attr.astype(F32), jnp.zeros((E, 5), F32)], axis=1)

    nf = _node_encoder(x2, posp, params["embedding"], params["node_in"])
    ef = _edge_encoder(eap, params["edge_in"])

    for lp in params["layers"]:
        w1 = lp["edge"]["lin"][0]["w"]
        pa, pb = _prep_stage(nf, w1[:128], w1[128:256])
        ga, gb = _sc_gather(pa, pb, dst3, src3)
        msg, ef = _edge_stage(ga, gb, ef, w1[256:], lp["edge"])
        agg2 = _sc_scatter(msg, dst3, zeros_n)
        nf = _node_stage(nf, agg2, lp["node"])

    out = _decoder_stage(nf, params["decoder"])
    return out[:, :2]


# speedup vs baseline: 3.5688x; 1.0010x over previous
import functools

import jax
import jax.numpy as jnp
from jax import lax
from jax.experimental import pallas as pl
from jax.experimental.pallas import tpu as pltpu
from jax.experimental.pallas import tpu_sc as plsc

F32 = jnp.float32
_NC, _NS = 2, 16
_NW = _NC * _NS
_NB = 4
_NBS = 3


def _pick(total, cap):
    for c in range(cap - cap % 8, 7, -8):
        if total % c == 0:
            return c
    raise ValueError((total, cap))


def _leaky(x):
    return jnp.where(x > 0, x, 0.01 * x)


def _ln(h, g, b):
    m = jnp.mean(h, axis=-1, keepdims=True)
    v = jnp.mean((h - m) ** 2, axis=-1, keepdims=True)
    return (h - m) / jnp.sqrt(v + 1e-5) * g + b


def _dot(a, b):
    return jnp.dot(a, b, preferred_element_type=F32)


def _v(x):
    return x.reshape(1, -1)


def _tile(br, bc):
    return pl.BlockSpec((br, bc), lambda i: (i, 0))


def _const(br, bc):
    return pl.BlockSpec((br, bc), lambda i: (0, 0))


def _edge_stage(ga, gb, ef, w1c, p):
    E = ef.shape[0]
    BE = _pick(E, 2048)

    def body(ga_r, gb_r, ef_r, w1c_r, w2_r, w3_r, w4_r,
             b1_r, b2_r, b3_r, b4_r, g_r, bn_r, msg_o, efn_o):
        ef_t = ef_r[...]
        h = ga_r[...] + gb_r[...] + _dot(ef_t, w1c_r[...]) + b1_r[...]
        h = _leaky(h)
        h = _leaky(_dot(h, w2_r[...]) + b2_r[...])
        h = _leaky(_dot(h, w3_r[...]) + b3_r[...])
        h = _dot(h, w4_r[...]) + b4_r[...]
        msg = _ln(h, g_r[...], bn_r[...])
        msg_o[...] = msg
        efn_o[...] = ef_t + msg

    ws = [w1c, p["lin"][1]["w"], p["lin"][2]["w"], p["lin"][3]["w"]]
    bs = [_v(p["lin"][i]["b"]) for i in range(4)]
    return pl.pallas_call(
        body,
        grid=(E // BE,),
        in_specs=[_tile(BE, 128)] * 3 + [_const(128, 128)] * 4
                 + [_const(1, 128)] * 6,
        out_specs=[_tile(BE, 128), _tile(BE, 128)],
        out_shape=[jax.ShapeDtypeStruct((E, 128), F32)] * 2,
    )(ga, gb, ef, *ws, *bs, _v(p["ln"]["g"]), _v(p["ln"]["b"]))


def _node_stage(nf, agg2, p):
    N = nf.shape[0]
    BN = _pick(N, 2048)
    v1 = p["lin"][0]["w"]

    def body(nf_r, a_r, v1a_r, v1b_r, w2_r, w3_r, w4_r,
             b1_r, b2_r, b3_r, b4_r, g_r, bn_r, out_o):
        nf_t = nf_r[...]
        a = a_r[0] + a_r[1]
        h = _dot(nf_t, v1a_r[...]) + _dot(a, v1b_r[...]) + b1_r[...]
        h = _leaky(h)
        h = _leaky(_dot(h, w2_r[...]) + b2_r[...])
        h = _leaky(_dot(h, w3_r[...]) + b3_r[...])
        h = _dot(h, w4_r[...]) + b4_r[...]
        out_o[...] = nf_t + _ln(h, g_r[...], bn_r[...])

    ws = [v1[:128], v1[128:], p["lin"][1]["w"], p["lin"][2]["w"],
          p["lin"][3]["w"]]
    bs = [_v(p["lin"][i]["b"]) for i in range(4)]
    return pl.pallas_call(
        body,
        grid=(N // BN,),
        in_specs=[_tile(BN, 128),
                  pl.BlockSpec((2, BN, 128), lambda i: (0, i, 0))]
                 + [_const(128, 128)] * 5 + [_const(1, 128)] * 6,
        out_specs=_tile(BN, 128),
        out_shape=jax.ShapeDtypeStruct((N, 128), F32),
    )(nf, agg2, *ws, *bs, _v(p["ln"]["g"]), _v(p["ln"]["b"]))


def _prep_stage(nf, w1a, w1b):
    N = nf.shape[0]
    BN = _pick(N, 2048)

    def body(nf_r, wa_r, wb_r, pa_o, pb_o):
        nf_t = nf_r[...]
        pa_o[...] = _dot(nf_t, wa_r[...])
        pb_o[...] = _dot(nf_t, wb_r[...])

    return pl.pallas_call(
        body,
        grid=(N // BN,),
        in_specs=[_tile(BN, 128)] + [_const(128, 128)] * 2,
        out_specs=[_tile(BN, 128), _tile(BN, 128)],
        out_shape=[jax.ShapeDtypeStruct((N, 128), F32)] * 2,
    )(nf, w1a, w1b)


def _node_encoder(x2, posp, emb, p):
    N = x2.shape[0]
    BN = _pick(N, 2048)
    w1 = p["lin"][0]["w"]
    wemb = jnp.zeros((16, 128), F32).at[:9].set(_dot(emb, w1[:16]))
    wpos = jnp.zeros((16, 128), F32).at[:14].set(w1[16:30])

    def body(x_r, pos_r, wemb_r, wpos_r, w2_r, w3_r, w4_r,
             b1_r, b2_r, b3_r, b4_r, g_r, bn_r, out_o):
        oh = (x_r[...] == lax.broadcasted_iota(jnp.int32, (BN, 16), 1))
        h = (_dot(oh.astype(F32), wemb_r[...]) + _dot(pos_r[...], wpos_r[...])
             + b1_r[...])
        h = _leaky(h)
        h = _leaky(_dot(h, w2_r[...]) + b2_r[...])
        h = _leaky(_dot(h, w3_r[...]) + b3_r[...])
        h = _dot(h, w4_r[...]) + b4_r[...]
        out_o[...] = _ln(h, g_r[...], bn_r[...])

    ws = [p["lin"][1]["w"], p["lin"][2]["w"], p["lin"][3]["w"]]
    bs = [_v(p["lin"][i]["b"]) for i in range(4)]
    return pl.pallas_call(
        body,
        grid=(N // BN,),
        in_specs=[pl.BlockSpec((BN, 1), lambda i: (i, 0)), _tile(BN, 16),
                  _const(16, 128), _const(16, 128)]
                 + [_const(128, 128)] * 3 + [_const(1, 128)] * 6,
        out_specs=_tile(BN, 128),
        out_shape=jax.ShapeDtypeStruct((N, 128), F32),
    )(x2, posp, wemb, wpos, *ws, *bs, _v(p["ln"]["g"]), _v(p["ln"]["b"]))


def _edge_encoder(eap, p):
    E = eap.shape[0]
    BE = _pick(E, 2048)
    w1p = jnp.zeros((8, 128), F32).at[:3].set(p["lin"][0]["w"])

    def body(ea_r, w1_r, w2_r, w3_r, w4_r,
             b1_r, b2_r, b3_r, b4_r, g_r, bn_r, out_o):
        h = _dot(ea_r[...], w1_r[...]) + b1_r[...]
        h = _leaky(h)
        h = _leaky(_dot(h, w2_r[...]) + b2_r[...])
        h = _leaky(_dot(h, w3_r[...]) + b3_r[...])
        h = _dot(h, w4_r[...]) + b4_r[...]
        out_o[...] = _ln(h, g_r[...], bn_r[...])

    ws = [p["lin"][1]["w"], p["lin"][2]["w"], p["lin"][3]["w"]]
    bs = [_v(p["lin"][i]["b"]) for i in range(4)]
    return pl.pallas_call(
        body,
        grid=(E // BE,),
        in_specs=[_tile(BE, 8), _const(8, 128)] + [_const(128, 128)] * 3
                 + [_const(1, 128)] * 6,
        out_specs=_tile(BE, 128),
        out_shape=jax.ShapeDtypeStruct((E, 128), F32),
    )(eap, w1p, *ws, *bs, _v(p["ln"]["g"]), _v(p["ln"]["b"]))


def _decoder_stage(nf, p):
    N = nf.shape[0]
    BN = _pick(N, 2048)
    w4p = jnp.zeros((128, 128), F32).at[:, :2].set(p["lin"][3]["w"])
    b4p = jnp.zeros((128,), F32).at[:2].set(p["lin"][3]["b"])

    def body(nf_r, w1_r, w2_r, w3_r, w4_r, b1_r, b2_r, b3_r, b4_r, out_o):
        h = _leaky(_dot(nf_r[...], w1_r[...]) + b1_r[...])
        h = _leaky(_dot(h, w2_r[...]) + b2_r[...])
        h = _leaky(_dot(h, w3_r[...]) + b3_r[...])
        out_o[...] = _dot(h, w4_r[...]) + b4_r[...]

    return pl.pallas_call(
        body,
        grid=(N // BN,),
        in_specs=[_tile(BN, 128)] + [_const(128, 128)] * 4
                 + [_const(1, 128)] * 4,
        out_specs=_tile(BN, 128),
        out_shape=jax.ShapeDtypeStruct((N, 128), F32),
    )(nf, p["lin"][0]["w"], p["lin"][1]["w"], p["lin"][2]["w"], w4p,
      _v(p["lin"][0]["b"]), _v(p["lin"][1]["b"]), _v(p["lin"][2]["b"]),
      _v(b4p))


def _sc_mesh():
    return plsc.VectorSubcoreMesh(core_axis_name="c", subcore_axis_name="s",
                                  num_cores=_NC, num_subcores=_NS)


def _sc_gather(pa, pb, dst3, src3):
    _, CH, K = dst3.shape
    EW = CH * K
    E = _NW * EW

    @functools.partial(
        pl.kernel,
        out_type=(jax.ShapeDtypeStruct((E, 128), F32),
                  jax.ShapeDtypeStruct((E, 128), F32)),
        mesh=_sc_mesh(),
        scratch_types=[
            pltpu.VMEM((CH, K), jnp.int32),
            pltpu.VMEM((CH, K), jnp.int32),
            pltpu.VMEM((_NB, K, 128), F32),
            pltpu.VMEM((_NB, K, 128), F32),
            pltpu.SemaphoreType.DMA((_NB,)),
            pltpu.SemaphoreType.DMA((_NB,)),
            pltpu.SemaphoreType.DMA((_NB,)),
            pltpu.SemaphoreType.DMA((_NB,)),
        ],
    )
    def k(pa_h, pb_h, d_h, s_h, ga_h, gb_h, di, si, ra, rb,
          sga, sgb, swa, swb):
        wid = lax.axis_index("s") * _NC + lax.axis_index("c")
        base = wid * EW
        pltpu.sync_copy(d_h.at[wid], di)
        pltpu.sync_copy(s_h.at[wid], si)
        pltpu.async_copy(pa_h.at[di.at[0]], ra.at[0], sga.at[0])
        pltpu.async_copy(pb_h.at[si.at[0]], rb.at[0], sgb.at[0])

        def body(j, carry):
            cur = lax.rem(j, _NB)
            nxt = lax.rem(j + 1, _NB)
            off = base + j * K
            pltpu.make_async_copy(pa_h.at[di.at[j]], ra.at[cur],
                                  sga.at[cur]).wait()
            pltpu.make_async_copy(pb_h.at[si.at[j]], rb.at[cur],
                                  sgb.at[cur]).wait()
            pltpu.async_copy(ra.at[cur], ga_h.at[pl.ds(off, K)], swa.at[cur])
            pltpu.async_copy(rb.at[cur], gb_h.at[pl.ds(off, K)], swb.at[cur])

            @pl.when(j >= _NB - 1)
            def _wait_oldest_writes():
                pltpu.make_async_copy(ra.at[nxt], ga_h.at[pl.ds(base, K)],
                                      swa.at[nxt]).wait()
                pltpu.make_async_copy(rb.at[nxt], gb_h.at[pl.ds(base, K)],
                                      swb.at[nxt]).wait()

            @pl.when(j + 1 < CH)
            def _start_next():
                pltpu.async_copy(pa_h.at[di.at[j + 1]], ra.at[nxt],
                                 sga.at[nxt])
                pltpu.async_copy(pb_h.at[si.at[j + 1]], rb.at[nxt],
                                 sgb.at[nxt])

            return carry

        lax.fori_loop(0, CH, body, 0)
        for j in range(max(CH - _NB + 1, 0), CH):
            s = j % _NB
            pltpu.make_async_copy(ra.at[s], ga_h.at[pl.ds(base, K)],
                                  swa.at[s]).wait()
            pltpu.make_async_copy(rb.at[s], gb_h.at[pl.ds(base, K)],
                                  swb.at[s]).wait()

    return k(pa, pb, dst3, src3)


def _sc_scatter(msg, dst3, zeros_n):
    _, CH, K = dst3.shape
    EW = CH * K
    N = zeros_n.shape[0]

    @functools.partial(
        pl.kernel,
        out_type=jax.ShapeDtypeStruct((_NC, N, 128), F32),
        mesh=_sc_mesh(),
        scratch_types=[
            pltpu.VMEM((CH, K), jnp.int32),
            pltpu.VMEM((_NBS, K, 128), F32),
            pltpu.VMEM_SHARED((N, 128), F32),
            pltpu.SemaphoreType.DMA((_NBS,)),
            pltpu.SemaphoreType.DMA((_NBS,)),
        ],
    )
    def k(m_h, d_h, z_h, out_h, di, mb, acc, sml, ssc):
        cid = lax.axis_index("c")
        sid = lax.axis_index("s")
        wid = sid * _NC + cid
        base = wid * EW
        NR = (N // (8 * _NS)) * 8
        roff = sid * NR

        @pl.when(sid < _NS - 1)
        def _zero():
            pltpu.sync_copy(z_h.at[pl.ds(roff, NR)], acc.at[pl.ds(roff, NR)])

        @pl.when(sid == _NS - 1)
        def _zero_last():
            NL = N - (_NS - 1) * NR
            pltpu.sync_copy(z_h.at[pl.ds((_NS - 1) * NR, NL)],
                            acc.at[pl.ds((_NS - 1) * NR, NL)])

        plsc.subcore_barrier()
        pltpu.sync_copy(d_h.at[wid], di)
        pltpu.async_copy(m_h.at[pl.ds(base, K)], mb.at[0], sml.at[0])

        def body(j, carry):
            cur = lax.rem(j, _NBS)
            nxt = lax.rem(j + 1, _NBS)
            pltpu.make_async_copy(m_h.at[pl.ds(base, K)], mb.at[cur],
                                  sml.at[cur]).wait()
            pltpu.async_copy(mb.at[cur], acc.at[di.at[j]], ssc.at[cur],
                             add=True)

            @pl.when(j >= _NBS - 1)
            def _wait_oldest_scatter():
                pltpu.make_async_copy(mb.at[nxt], acc.at[di.at[0]],
                                      ssc.at[nxt]).wait()

            @pl.when(j + 1 < CH)
            def _start_next():
                pltpu.async_copy(m_h.at[pl.ds(base + (j + 1) * K, K)],
                                 mb.at[nxt], sml.at[nxt])

            return carry

        lax.fori_loop(0, CH, body, 0)
        for j in range(max(CH - _NBS + 1, 0), CH):
            s = j % _NBS
            pltpu.make_async_copy(mb.at[s], acc.at[di.at[0]],
                                  ssc.at[s]).wait()
        plsc.subcore_barrier()

        @pl.when(sid < _NS - 1)
        def _dump():
            pltpu.sync_copy(acc.at[pl.ds(roff, NR)],
                            out_h.at[cid].at[pl.ds(roff, NR)])

        @pl.when(sid == _NS - 1)
        def _dump_last():
            NL = N - (_NS - 1) * NR
            pltpu.sync_copy(acc.at[pl.ds((_NS - 1) * NR, NL)],
                            out_h.at[cid].at[pl.ds((_NS - 1) * NR, NL)])

    return k(msg, dst3, zeros_n)


def kernel(x, pos, edge_index, edge_attr, params):
    N = pos.shape[0]
    E = edge_attr.shape[0]
    EW = E // _NW
    K = _pick(EW, 128)
    CH = EW // K

    src = edge_index[0].astype(jnp.int32)
    dst = edge_index[1].astype(jnp.int32)
    dst3 = dst.reshape(_NW, CH, K)
    src3 = src.reshape(_NW, CH, K)
    zeros_n = jnp.zeros((N, 128), F32)

    x2 = x.astype(jnp.int32).reshape(N, 1)
    posp = jnp.concatenate(
        [pos.astype(F32), jnp.zeros((N, 2), F32)], axis=1)
    eap = jnp.concatenate(
        [edge_attr.astype(F32), jnp.zeros((E, 5), F32)], axis=1)

    nf = _node_encoder(x2, posp, params["embedding"], params["node_in"])
    ef = _edge_encoder(eap, params["edge_in"])

    for lp in params["layers"]:
        w1 = lp["edge"]["lin"][0]["w"]
        pa, pb = _prep_stage(nf, w1[:128], w1[128:256])
        ga, gb = _sc_gather(pa, pb, dst3, src3)
        msg, ef = _edge_stage(ga, gb, ef, w1[256:], lp["edge"])
        agg2 = _sc_scatter(msg, dst3, zeros_n)
        nf = _node_stage(nf, agg2, lp["node"])

    out = _decoder_stage(nf, params["decoder"])
    return out[:, :2]
